# baseline (speedup 1.0000x reference)
import functools

import jax
import jax.numpy as jnp
import numpy as np
from jax import lax
from jax.experimental import pallas as pl
from jax.experimental.pallas import tpu as pltpu
from jax.experimental.pallas import tpu_sc as plsc

F32 = jnp.float32

B = 2
NPB = 25000
N = B * NPB
E = 800000
COUNTERS = 128
HID = 64
H2 = 32

NC = 2
NS = 16
NW = NC * NS
SUB = 125
SUBS = 4
STAGE = SUB * SUBS
EH = E // 2
ROWS3 = E // SUB
STAGES_H = ROWS3 // SUBS // 2
STAGES_W = STAGES_H // NW
STAGES_C = STAGES_H // NC // NS
NPT = N // NS


def _mm(a, b):
    return lax.dot_general(a, b, (((a.ndim - 1,), (0,)), ((), ())))


def _leaky(v):
    return jnp.where(v >= 0, v, 0.01 * v)


def _swish(v):
    return v * jax.nn.sigmoid(v)


_MESH = plsc.VectorSubcoreMesh(core_axis_name="c", subcore_axis_name="s")


def _sc_gather_body(half, a_hbm, c_hbm, dst3_hbm, src3_hbm, pre_hbm,
                    didx, sidx, bufa, bufb, bufo, sem):
    cid = lax.axis_index("c")
    sid = lax.axis_index("s")
    w = cid * NS + sid
    base = half * STAGES_H + w * STAGES_W

    def issue(st_r0):
        pltpu.sync_copy(dst3_hbm.at[pl.ds(st_r0, SUBS)], didx)
        pltpu.sync_copy(src3_hbm.at[pl.ds(st_r0, SUBS)], sidx)
        for j in range(SUBS):
            pltpu.async_copy(
                a_hbm.at[didx.at[j]], bufa.at[pl.ds(j * SUB, SUB)], sem)
            pltpu.async_copy(
                c_hbm.at[sidx.at[j]], bufb.at[pl.ds(j * SUB, SUB)], sem)

    issue(base * SUBS)

    def body(t, carry):
        for j in range(SUBS):
            pltpu.make_async_copy(
                a_hbm.at[didx.at[j]], bufa.at[pl.ds(j * SUB, SUB)], sem).wait()
            pltpu.make_async_copy(
                c_hbm.at[sidx.at[j]], bufb.at[pl.ds(j * SUB, SUB)], sem).wait()

        def add_body(i, c2):
            for rr in range(4):
                r = i * 4 + rr
                for hh in range(2):
                    sl = pl.ds(hh * 16, 16)
                    bufo[r, sl] = bufa[r, sl] + bufb[r, sl]
            return c2

        lax.fori_loop(0, STAGE // 4, add_body, 0)

        @pl.when(t < STAGES_W - 1)
        def _():
            issue((base + t + 1) * SUBS)

        e0 = (base + t) * STAGE - half * EH
        pltpu.sync_copy(bufo, pre_hbm.at[pl.ds(e0, STAGE)])
        return carry

    lax.fori_loop(0, STAGES_W, body, 0)


_SC_PARAMS = pltpu.CompilerParams(use_tc_tiling_on_sc=False)

_sc_gather = [pl.kernel(
    functools.partial(_sc_gather_body, h),
    out_type=jax.ShapeDtypeStruct((EH, H2), F32),
    mesh=_MESH,
    compiler_params=_SC_PARAMS,
    scratch_types=[
        pltpu.VMEM((SUBS, SUB), jnp.int32),
        pltpu.VMEM((SUBS, SUB), jnp.int32),
        pltpu.VMEM((STAGE, H2), F32),
        pltpu.VMEM((STAGE, H2), F32),
        pltpu.VMEM((STAGE, H2), F32),
        pltpu.SemaphoreType.DMA,
    ],
) for h in range(2)]


def _sc_scatter_body(half, m_hbm, dst3_hbm, zz_hbm, out_hbm, didx, mbuf, acc):
    cid = lax.axis_index("c")
    sid = lax.axis_index("s")
    pltpu.sync_copy(zz_hbm.at[pl.ds(sid * NPT, NPT)],
                    acc.at[pl.ds(sid * NPT, NPT)])
    plsc.subcore_barrier()
    spc = STAGES_C * NS

    def body(t, carry):
        st = half * STAGES_H + cid * spc + sid * STAGES_C + t
        r0 = st * SUBS
        e0 = st * STAGE - half * EH
        pltpu.sync_copy(dst3_hbm.at[pl.ds(r0, SUBS)], didx)
        pltpu.sync_copy(m_hbm.at[pl.ds(e0, STAGE)], mbuf)
        for j in range(SUBS):
            pltpu.sync_copy(mbuf.at[pl.ds(j * SUB, SUB)],
                            acc.at[didx.at[j]], add=True)
        return carry

    lax.fori_loop(0, STAGES_C, body, 0)
    plsc.subcore_barrier()
    pltpu.sync_copy(acc.at[pl.ds(sid * NPT, NPT)],
                    out_hbm.at[cid, pl.ds(sid * NPT, NPT)])


_sc_scatter = [pl.kernel(
    functools.partial(_sc_scatter_body, h),
    out_type=jax.ShapeDtypeStruct((NC, N, H2), F32),
    mesh=_MESH,
    compiler_params=_SC_PARAMS,
    scratch_types=[
        pltpu.VMEM((SUBS, SUB), jnp.int32),
        pltpu.VMEM((STAGE, H2), F32),
        pltpu.VMEM_SHARED((N, H2), F32),
    ],
) for h in range(2)]


def _sc_count_body(dst3_hbm, ones_hbm, zz_hbm, out_hbm, didx, obuf, acc):
    cid = lax.axis_index("c")
    sid = lax.axis_index("s")
    pltpu.sync_copy(zz_hbm.at[pl.ds(sid * NPT, NPT)],
                    acc.at[pl.ds(sid * NPT, NPT)])
    pltpu.sync_copy(ones_hbm, obuf)
    plsc.subcore_barrier()
    spc = STAGES_C * NS * 2

    def body(t, carry):
        st = cid * spc + sid * STAGES_C * 2 + t
        r0 = st * SUBS
        pltpu.sync_copy(dst3_hbm.at[pl.ds(r0, SUBS)], didx)
        for j in range(SUBS):
            pltpu.sync_copy(obuf, acc.at[didx.at[j]], add=True)
        return carry

    lax.fori_loop(0, STAGES_C * 2, body, 0)
    plsc.subcore_barrier()
    pltpu.sync_copy(acc.at[pl.ds(sid * NPT, NPT)],
                    out_hbm.at[cid, pl.ds(sid * NPT, NPT)])


_sc_count = pl.kernel(
    _sc_count_body,
    out_type=jax.ShapeDtypeStruct((NC, N, H2), F32),
    mesh=_MESH,
    compiler_params=_SC_PARAMS,
    scratch_types=[
        pltpu.VMEM((SUBS, SUB), jnp.int32),
        pltpu.VMEM((SUB, H2), F32),
        pltpu.VMEM_SHARED((N, H2), F32),
    ],
)


HBLK = 2048
HGRID = -(-N // HBLK)


def _head_body(x_ref, num_ref, cc_ref, y_ref, bsel_ref,
               imp_ref, one_ref, tun_ref, lan_ref, wc_ref, bc_ref,
               w1_ref, b1_ref, w2_ref, b2_ref, wpt_ref, bp_ref,
               wout_ref, bout_ref, w1h_ref, w1ea_ref, bemb_ref,
               out_ref):
    wc = wc_ref[...]
    bc = bc_ref[...]
    t_imp = imp_ref[...]
    t_one = one_ref[...]
    t_tun = tun_ref[...]
    t_lan = lan_ref[...]
    base = (t_imp[0:1] @ wc[0:5] + t_one[0:1] @ wc[5:7]
            + t_tun[0:1] @ wc[7:9] + t_lan[0:1] @ wc[9:12] + bc)
    d_imp = (t_imp[1:2] - t_imp[0:1]) @ wc[0:5]
    d_one = (t_one[1:2] - t_one[0:1]) @ wc[5:7]
    d_tun = (t_tun[1:2] - t_tun[0:1]) @ wc[7:9]
    d_lan = (t_lan[1:2] - t_lan[0:1]) @ wc[9:12]
    cc = cc_ref[...].astype(F32)
    num = num_ref[...]
    y0 = y_ref[...]
    ea = (base + cc[:, 0:1] * d_imp + cc[:, 1:2] * d_one
          + cc[:, 2:3] * d_tun + cc[:, 3:4] * d_lan
          + _mm(num, wc[12:20]) + y0 * wc[20:21])
    ea = _leaky(ea)

    xv = x_ref[...]
    h = jax.nn.relu(_mm(xv, w1_ref[...]) + b1_ref[...])
    h = jax.nn.relu(_mm(h, w2_ref[...]) + b2_ref[...])
    t2 = lax.dot_general(wpt_ref[...], h, (((1,), (1,)), ((), ())))
    t2 = t2 + bp_ref[...]
    bs = bsel_ref[...]
    hcol = t2[:, 0:1] * (1.0 - bs) + t2[:, 1:2] * bs

    v = wout_ref[...] @ w1h_ref[...]
    c0 = bout_ref[...] @ w1h_ref[...] + bemb_ref[...]
    out_ref[...] = hcol * v + _mm(ea, w1ea_ref[...]) + c0


E4 = EH // 4
N4 = N // 4
EBLK = 2000
EGRID = E4 // EBLK


def _mid_body(pre_ref, g_ref, bb_ref, w2_ref, b2_ref, out_ref):
    s = _swish(pre_ref[...]) * g_ref[...] + bb_ref[...]
    out_ref[...] = _swish(_mm(s, w2_ref[...]) + b2_ref[...])


UBLK = 512
UGRID = -(-N4 // UBLK)


def _update_body(xx_ref, ppa_ref, ppb_ref, cn_ref, u1t_ref, u1b_ref, bu1_ref,
                 u2_ref, bu2_ref, wd_ref, bm_ref, ws_ref,
                 out_ref, a_ref, c_ref):
    xx = xx_ref[...]
    p = ppa_ref[0] + ppa_ref[1] + ppb_ref[0] + ppb_ref[1]
    c = cn_ref[0] + cn_ref[1]
    agg = p * (1.0 / jnp.maximum(c, 1.0))
    u = _swish(_mm(xx, u1t_ref[...]) + _mm(agg, u1b_ref[...]) + bu1_ref[...])
    u = _swish(_mm(u, u2_ref[...]) + bu2_ref[...])
    xxn = xx + u
    out_ref[...] = xxn
    a_ref[...] = _mm(xxn, wd_ref[...]) + bm_ref[...]
    c_ref[...] = _mm(xxn, ws_ref[...])


def _ac_body(xx_ref, wd_ref, bm_ref, ws_ref, a_ref, c_ref):
    xx = xx_ref[...]
    a_ref[...] = _mm(xx, wd_ref[...]) + bm_ref[...]
    c_ref[...] = _mm(xx, ws_ref[...])


def _tail_body(xx_ref, ppa_ref, ppb_ref, cn_ref, u1t_ref, u1b_ref, bu1_ref,
               u2_ref, bu2_ref, we_ref, be_ref,
               wa0_ref, ba0_ref, wb0_ref, bb0_ref, wf0_ref, bf0_ref,
               wa1_ref, ba1_ref, wb1_ref, bb1_ref, wf1_ref, bf1_ref,
               wa2_ref, ba2_ref, wb2_ref, bb2_ref, wf2_ref, bf2_ref,
               o0_ref, o1_ref, o2_ref):
    xx = xx_ref[...]
    p = ppa_ref[0] + ppa_ref[1] + ppb_ref[0] + ppb_ref[1]
    c = cn_ref[0] + cn_ref[1]
    agg = p * (1.0 / jnp.maximum(c, 1.0))
    u = _swish(_mm(xx, u1t_ref[...]) + _mm(agg, u1b_ref[...]) + bu1_ref[...])
    u = _swish(_mm(u, u2_ref[...]) + bu2_ref[...])
    xx = xx + u
    y = _mm(xx, we_ref[...]) + be_ref[...]

    def head(wa, ba, wb, bb, wf, bf):
        h = _mm(y, wa) + ba
        h = h + _leaky(h)
        h = _mm(h, wb) + bb
        h = h + _leaky(h)
        return _mm(h, wf) + bf

    o0_ref[...] = head(wa0_ref[...], ba0_ref[...], wb0_ref[...], bb0_ref[...],
                       wf0_ref[...], bf0_ref[...])
    o1_ref[...] = head(wa1_ref[...], ba1_ref[...], wb1_ref[...], bb1_ref[...],
                       wf1_ref[...], bf1_ref[...])
    o2_ref[...] = head(wa2_ref[...], ba2_ref[...], wb2_ref[...], bb2_ref[...],
                       wf2_ref[...], bf2_ref[...])


def _wspec(shape):
    rank = len(shape)
    return pl.BlockSpec(shape, lambda k: (0,) * rank)


def _r2(b):
    return jnp.reshape(b, (1, -1))


def kernel(x, num_attr, cc_attr, y_init, edge_index, params):
    p = params
    src = edge_index[0]
    dst = edge_index[1]
    dst3 = dst.reshape(ROWS3, SUB)
    src3 = src.reshape(ROWS3, SUB)
    zz = np.zeros((N, H2), np.float32)
    ones_t = np.ones((SUB, H2), np.float32)

    inv_std = 1.0 / jnp.sqrt(1.0 + 1e-5)

    head_call = pl.pallas_call(
        _head_body,
        grid=(HGRID,),
        in_specs=[
            _wspec((B, COUNTERS)),
            pl.BlockSpec((HBLK, 8), lambda k: (k, 0)),
            pl.BlockSpec((HBLK, 4), lambda k: (k, 0)),
            pl.BlockSpec((HBLK, 1), lambda k: (k, 0)),
            pl.BlockSpec((HBLK, 1), lambda k: (k, 0)),
            _wspec((8, 5)), _wspec((2, 2)), _wspec((2, 2)), _wspec((6, 3)),
            _wspec((21, HID)), _wspec((1, HID)),
            _wspec((COUNTERS, COUNTERS)), _wspec((1, COUNTERS)),
            _wspec((COUNTERS, COUNTERS)), _wspec((1, COUNTERS)),
            pl.BlockSpec((HBLK, COUNTERS), lambda k: (k, 0)),
            pl.BlockSpec((HBLK, 1), lambda k: (k, 0)),
            _wspec((1, 3)), _wspec((1, 3)), _wspec((3, H2)),
            _wspec((HID, H2)), _wspec((1, H2)),
        ],
        out_specs=pl.BlockSpec((HBLK, H2), lambda k: (k, 0)),
        out_shape=jax.ShapeDtypeStruct((N, H2), F32),
    )
    W1, b1 = p['emb1']
    bsel = (np.arange(N) >= NPB).astype(np.float32).reshape(N, 1)
    xx0 = head_call(
        x, num_attr.reshape(N, 8), cc_attr.reshape(N, 4),
        y_init.reshape(N, 1), bsel,
        p['emb_imp'], p['emb_one'], p['emb_tun'], p['emb_lan'],
        p['coords'][0], _r2(p['coords'][1]),
        p['mlp_h1'][0], _r2(p['mlp_h1'][1]),
        p['mlp_h2'][0], _r2(p['mlp_h2'][1]),
        jnp.tile(p['mlp_pred'][0].T, (B, 1)),
        jnp.tile(p['mlp_pred'][1].reshape(NPB, 1), (B, 1)),
        p['mlp_out'][0], _r2(p['mlp_out'][1]),
        W1[:3], W1[3:], _r2(b1),
    )
    xxp = xx0.reshape(N4, 128)

    cnt2 = _sc_count(dst3, ones_t, zz)
    cnp = cnt2.reshape(NC, N4, 128)
    cn1 = cnt2[:, :, :1]

    def _bd(w):
        return jnp.kron(jnp.eye(4, dtype=F32), w)

    def _b4(b):
        return jnp.tile(b.reshape(1, H2), (1, 4))

    mid_call = pl.pallas_call(
        _mid_body,
        grid=(EGRID,),
        in_specs=[
            pl.BlockSpec((EBLK, 128), lambda k: (k, 0)),
            _wspec((1, 128)), _wspec((1, 128)),
            _wspec((128, 128)), _wspec((1, 128)),
        ],
        out_specs=pl.BlockSpec((EBLK, 128), lambda k: (k, 0)),
        out_shape=jax.ShapeDtypeStruct((E4, 128), F32),
    )

    _nspec = pl.BlockSpec((UBLK, 128), lambda k: (k, 0))
    _nshape = jax.ShapeDtypeStruct((N4, 128), F32)
    update_call = pl.pallas_call(
        _update_body,
        grid=(UGRID,),
        in_specs=[
            _nspec,
            pl.BlockSpec((NC, UBLK, 128), lambda k: (0, k, 0)),
            pl.BlockSpec((NC, UBLK, 128), lambda k: (0, k, 0)),
            pl.BlockSpec((NC, UBLK, 128), lambda k: (0, k, 0)),
            _wspec((128, 128)), _wspec((128, 128)), _wspec((1, 128)),
            _wspec((128, 128)), _wspec((1, 128)),
            _wspec((128, 128)), _wspec((1, 128)), _wspec((128, 128)),
        ],
        out_specs=[_nspec, _nspec, _nspec],
        out_shape=[_nshape, _nshape, _nshape],
    )
    ac_call = pl.pallas_call(
        _ac_body,
        grid=(UGRID,),
        in_specs=[_nspec, _wspec((128, 128)), _wspec((1, 128)),
                  _wspec((128, 128))],
        out_specs=[_nspec, _nspec],
        out_shape=[_nshape, _nshape],
    )

    def _m1w(lp):
        W_m1, b_m1 = lp['m1']
        return _bd(W_m1[:H2]), _b4(b_m1), _bd(W_m1[H2:])

    ap, cp = ac_call(xxp, *_m1w(p['gnn'][0]))
    ppA = ppB = None
    for li, lp in enumerate(p['gnn']):
        mid_w = (_b4(inv_std * lp['bn_g']), _b4(lp['bn_b']),
                 _bd(lp['m2'][0]), _b4(lp['m2'][1]))
        a_n = ap.reshape(N, H2)
        c_n = cp.reshape(N, H2)
        pre0 = _sc_gather[0](a_n, c_n, dst3, src3)
        pre1 = _sc_gather[1](a_n, c_n, dst3, src3)
        m0 = mid_call(pre0.reshape(E4, 128), *mid_w)
        m1 = mid_call(pre1.reshape(E4, 128), *mid_w)
        ppA = _sc_scatter[0](m0.reshape(EH, H2), dst3, zz)
        ppB = _sc_scatter[1](m1.reshape(EH, H2), dst3, zz)
        if li < 3:
            W_u1, b_u1 = lp['u1']
            xxp, ap, cp = update_call(
                xxp, ppA.reshape(NC, N4, 128), ppB.reshape(NC, N4, 128), cnp,
                _bd(W_u1[:H2]), _bd(W_u1[H2:]), _b4(b_u1),
                _bd(lp['u2'][0]), _b4(lp['u2'][1]),
                *_m1w(p['gnn'][li + 1]),
            )

    lp = p['gnn'][3]
    W_u1, b_u1 = lp['u1']
    tail_call = pl.pallas_call(
        _tail_body,
        grid=(HGRID,),
        in_specs=[
            pl.BlockSpec((HBLK, H2), lambda k: (k, 0)),
            pl.BlockSpec((NC, HBLK, H2), lambda k: (0, k, 0)),
            pl.BlockSpec((NC, HBLK, H2), lambda k: (0, k, 0)),
            pl.BlockSpec((NC, HBLK, 1), lambda k: (0, k, 0)),
            _wspec((H2, H2)), _wspec((H2, H2)), _wspec((1, H2)),
            _wspec((H2, H2)), _wspec((1, H2)),
            _wspec((H2, HID)), _wspec((1, HID)),
            _wspec((HID, HID)), _wspec((1, HID)),
            _wspec((HID, HID)), _wspec((1, HID)),
            _wspec((HID, 3)), _wspec((1, 3)),
            _wspec((HID, HID)), _wspec((1, HID)),
            _wspec((HID, HID)), _wspec((1, HID)),
            _wspec((HID, 1)), _wspec((1, 1)),
            _wspec((HID, HID)), _wspec((1, HID)),
            _wspec((HID, HID)), _wspec((1, HID)),
            _wspec((HID, 3)), _wspec((1, 3)),
        ],
        out_specs=[
            pl.BlockSpec((HBLK, 3), lambda k: (k, 0)),
            pl.BlockSpec((HBLK, 1), lambda k: (k, 0)),
            pl.BlockSpec((HBLK, 3), lambda k: (k, 0)),
        ],
        out_shape=[
            jax.ShapeDtypeStruct((N, 3), F32),
            jax.ShapeDtypeStruct((N, 1), F32),
            jax.ShapeDtypeStruct((N, 3), F32),
        ],
    )
    h0, h1, h2w = p['pred0'], p['pred1'], p['pred2']
    o0, o1, o2 = tail_call(
        xxp.reshape(N, H2), ppA, ppB, cn1, W_u1[:H2], W_u1[H2:], _r2(b_u1),
        lp['u2'][0], _r2(lp['u2'][1]),
        p['emb2'][0], _r2(p['emb2'][1]),
        h0[0][0], _r2(h0[0][1]), h0[1][0], _r2(h0[1][1]), h0[2][0], _r2(h0[2][1]),
        h1[0][0], _r2(h1[0][1]), h1[1][0], _r2(h1[1][1]), h1[2][0], _r2(h1[2][1]),
        h2w[0][0], _r2(h2w[0][1]), h2w[1][0], _r2(h2w[1][1]), h2w[2][0], _r2(h2w[2][1]),
    )
    return (o0.reshape(B, NPB, 3), o1.reshape(B, NPB, 1), o2.reshape(B, NPB, 3))
